# baseline (speedup 1.0000x reference)
import functools

import jax
import jax.numpy as jnp
from jax import lax
from jax.experimental import pallas as pl
from jax.experimental.pallas import tpu as pltpu
from jax.experimental.pallas import tpu_sc as plsc

NC = 2
NS = 16
NW = NC * NS
LANES = 16

C = 40
IDX_W = 80


def _agg_body(S, D, NCH, features_hbm, tn_hbm, scale_hbm, out_hbm,
              idx0, idx1, rows0, rows1, out0, out1, scale_v,
              gsem0, gsem1, isem0, isem1, osem0, osem1):
    k_slices = (C * S) // IDX_W
    g_regs = D // (2 * LANES)

    wid = lax.axis_index("s") * NC + lax.axis_index("c")

    pltpu.sync_copy(scale_hbm, scale_v)
    scale = scale_v[...]

    idx = (idx0, idx1)
    rows = (rows0, rows1)
    outs = (out0, out1)
    gsem = (gsem0, gsem1)
    isem = (isem0, isem1)
    osem = (osem0, osem1)

    def idx_copy(c, p):
        return pltpu.make_async_copy(tn_hbm.at[c], idx[p], isem[p])

    def gather_copies(p):
        return [
            pltpu.make_async_copy(
                features_hbm.at[idx[p].at[j]],
                rows[p].at[pl.ds(j * IDX_W, IDX_W)], gsem[p])
            for j in range(k_slices)
        ]

    def out_copy(c, p):
        return pltpu.make_async_copy(outs[p], out_hbm.at[pl.ds(c * C, C)],
                                     osem[p])

    evens = 2 * lax.iota(jnp.int32, LANES)
    hi_mask = jnp.full((LANES,), -65536, jnp.int32)

    def compute(p):
        @plsc.parallel_loop(0, C, unroll=4)
        def node_body(n):
            r = n * S
            row_idx = jnp.full((LANES,), n, jnp.int32)
            for g in range(g_regs):
                sl = pl.ds(g * LANES, LANES)
                acc_e = None
                acc_o = None
                for s in range(S):
                    v = rows[p][r + s, sl]
                    lo = plsc.bitcast(lax.shift_left(v, 16), jnp.float32)
                    hi = plsc.bitcast(jnp.bitwise_and(v, hi_mask),
                                      jnp.float32)
                    acc_e = lo if acc_e is None else acc_e + lo
                    acc_o = hi if acc_o is None else acc_o + hi
                cols = evens + (2 * LANES * g)
                plsc.store_scatter(outs[p], [row_idx, cols], acc_e * scale)
                plsc.store_scatter(outs[p], [row_idx, cols + 1],
                                   acc_o * scale)

    @pl.when(wid < NCH)
    def _():
        idx_copy(wid, 0).start()
        idx_copy(wid, 0).wait()
        for cp in gather_copies(0):
            cp.start()

    @pl.when(wid + NW < NCH)
    def _():
        idx_copy(wid + NW, 1).start()

    n_pairs = (-(-NCH // NW) + 1) // 2

    def pair_body(i2, _):
        for p in (0, 1):
            t = 2 * i2 + p
            c = wid + t * NW
            q = 1 - p

            @pl.when(c + NW < NCH)
            def _():
                idx_copy(c + NW, q).wait()
                for cp in gather_copies(q):
                    cp.start()

            @pl.when(c < NCH)
            def _():
                for cp in gather_copies(p):
                    cp.wait()

                @pl.when(c + 2 * NW < NCH)
                def _():
                    idx_copy(c + 2 * NW, p).start()

                @pl.when(i2 >= 1)
                def _():
                    out_copy(c - 2 * NW, p).wait()

                compute(p)
                out_copy(c, p).start()
        return 0

    lax.fori_loop(0, n_pairs, pair_body, 0, unroll=False)

    for p in (0, 1):
        @pl.when(wid + p * NW < NCH)
        def _():
            out_copy(0, p).wait()


def kernel(features, nodes, to_neighs, num_sample):
    del nodes
    B, S = to_neighs.shape
    N, D = features.shape
    assert B % C == 0 and (C * S) % IDX_W == 0 and D % (2 * LANES) == 0
    NCH = B // C

    fb = lax.bitcast_convert_type(
        features.astype(jnp.bfloat16).reshape(N, D // 2, 2), jnp.int32)
    tn = to_neighs.reshape(NCH, (C * S) // IDX_W, IDX_W)
    scale = jnp.full((LANES,), 1.0, jnp.float32) / num_sample

    mesh = plsc.VectorSubcoreMesh(
        core_axis_name="c", subcore_axis_name="s",
        num_cores=NC, num_subcores=NS)
    k_sl = (C * S) // IDX_W
    grid_kernel = functools.partial(
        pl.kernel,
        out_type=jax.ShapeDtypeStruct((B, D), jnp.float32),
        mesh=mesh,
        compiler_params=pltpu.CompilerParams(
            use_tc_tiling_on_sc=False, needs_layout_passes=False),
        scratch_types=[
            pltpu.VMEM((k_sl, IDX_W), jnp.int32),
            pltpu.VMEM((k_sl, IDX_W), jnp.int32),
            pltpu.VMEM((C * S, D // 2), jnp.int32),
            pltpu.VMEM((C * S, D // 2), jnp.int32),
            pltpu.VMEM((C, D), jnp.float32),
            pltpu.VMEM((C, D), jnp.float32),
            pltpu.VMEM((LANES,), jnp.float32),
            pltpu.SemaphoreType.DMA,
            pltpu.SemaphoreType.DMA,
            pltpu.SemaphoreType.DMA,
            pltpu.SemaphoreType.DMA,
            pltpu.SemaphoreType.DMA,
            pltpu.SemaphoreType.DMA,
        ],
    )(functools.partial(_agg_body, S, D, NCH))
    return grid_kernel(fb, tn, scale)


# speedup vs baseline: 2.0835x; 2.0835x over previous
import functools

import jax
import jax.numpy as jnp
from jax import lax
from jax.experimental import pallas as pl
from jax.experimental.pallas import tpu as pltpu
from jax.experimental.pallas import tpu_sc as plsc

NC = 2
NS = 16
NW = NC * NS
LANES = 16

C = 40
IDX_W = 80


def _agg_body(S, D, NCH, features_hbm, tn_hbm, scale_hbm, out_hbm,
              idx0, idx1, rows0, rows1, out0, out1, scale_v,
              gsem0, gsem1, isem0, isem1, osem0, osem1):
    k_slices = (C * S) // IDX_W
    g_regs = D // (2 * LANES)

    wid = lax.axis_index("s") * NC + lax.axis_index("c")

    pltpu.sync_copy(scale_hbm, scale_v)
    scale = scale_v[...]

    idx = (idx0, idx1)
    rows = (rows0, rows1)
    outs = (out0, out1)
    gsem = (gsem0, gsem1)
    isem = (isem0, isem1)
    osem = (osem0, osem1)

    def idx_copy(c, p):
        return pltpu.make_async_copy(tn_hbm.at[c], idx[p], isem[p])

    def gather_copies(p):
        return [
            pltpu.make_async_copy(
                features_hbm.at[idx[p].at[j]],
                rows[p].at[pl.ds(j * IDX_W, IDX_W)], gsem[p])
            for j in range(k_slices)
        ]

    def out_copy(c, p):
        return pltpu.make_async_copy(outs[p], out_hbm.at[pl.ds(c * C, C)],
                                     osem[p])

    hi_mask = jnp.full((LANES,), -65536, jnp.int32)

    def compute(p):
        @plsc.parallel_loop(0, C, unroll=4)
        def node_body(n):
            r = n * S
            half = D // 2
            for g in range(g_regs):
                sl = pl.ds(g * LANES, LANES)
                acc_lo = None
                acc_hi = None
                for s in range(S):
                    v = rows[p][r + s, sl]
                    lo = plsc.bitcast(lax.shift_left(v, 16), jnp.float32)
                    hi = plsc.bitcast(jnp.bitwise_and(v, hi_mask),
                                      jnp.float32)
                    acc_lo = lo if acc_lo is None else acc_lo + lo
                    acc_hi = hi if acc_hi is None else acc_hi + hi
                outs[p][n, pl.ds(g * LANES, LANES)] = acc_lo * scale
                outs[p][n, pl.ds(half + g * LANES, LANES)] = acc_hi * scale

    @pl.when(wid < NCH)
    def _():
        idx_copy(wid, 0).start()
        idx_copy(wid, 0).wait()
        for cp in gather_copies(0):
            cp.start()

    @pl.when(wid + NW < NCH)
    def _():
        idx_copy(wid + NW, 1).start()

    n_pairs = (-(-NCH // NW) + 1) // 2

    def pair_body(i2, _):
        for p in (0, 1):
            t = 2 * i2 + p
            c = wid + t * NW
            q = 1 - p

            @pl.when(c + NW < NCH)
            def _():
                idx_copy(c + NW, q).wait()
                for cp in gather_copies(q):
                    cp.start()

            @pl.when(c < NCH)
            def _():
                for cp in gather_copies(p):
                    cp.wait()

                @pl.when(c + 2 * NW < NCH)
                def _():
                    idx_copy(c + 2 * NW, p).start()

                @pl.when(i2 >= 1)
                def _():
                    out_copy(c - 2 * NW, p).wait()

                compute(p)
                out_copy(c, p).start()
        return 0

    lax.fori_loop(0, n_pairs, pair_body, 0, unroll=False)

    for p in (0, 1):
        @pl.when(wid + p * NW < NCH)
        def _():
            out_copy(0, p).wait()


def kernel(features, nodes, to_neighs, num_sample):
    del nodes
    B, S = to_neighs.shape
    N, D = features.shape
    assert B % C == 0 and (C * S) % IDX_W == 0 and D % (2 * LANES) == 0
    NCH = B // C

    u16 = jnp.uint32(16)
    bits = lax.bitcast_convert_type(features, jnp.uint32)
    rne = (bits + jnp.uint32(0x7FFF)
           + (lax.shift_right_logical(bits, u16) & jnp.uint32(1)))
    top = lax.shift_right_logical(rne, u16)
    fb = lax.bitcast_convert_type(
        top[:, :D // 2] | lax.shift_left(top[:, D // 2:], u16), jnp.int32)
    tn = to_neighs.reshape(NCH, (C * S) // IDX_W, IDX_W)
    scale = jnp.full((LANES,), 1.0, jnp.float32) / num_sample

    mesh = plsc.VectorSubcoreMesh(
        core_axis_name="c", subcore_axis_name="s",
        num_cores=NC, num_subcores=NS)
    k_sl = (C * S) // IDX_W
    grid_kernel = functools.partial(
        pl.kernel,
        out_type=jax.ShapeDtypeStruct((B, D), jnp.float32),
        mesh=mesh,
        compiler_params=pltpu.CompilerParams(
            use_tc_tiling_on_sc=False, needs_layout_passes=False),
        scratch_types=[
            pltpu.VMEM((k_sl, IDX_W), jnp.int32),
            pltpu.VMEM((k_sl, IDX_W), jnp.int32),
            pltpu.VMEM((C * S, D // 2), jnp.int32),
            pltpu.VMEM((C * S, D // 2), jnp.int32),
            pltpu.VMEM((C, D), jnp.float32),
            pltpu.VMEM((C, D), jnp.float32),
            pltpu.VMEM((LANES,), jnp.float32),
            pltpu.SemaphoreType.DMA,
            pltpu.SemaphoreType.DMA,
            pltpu.SemaphoreType.DMA,
            pltpu.SemaphoreType.DMA,
            pltpu.SemaphoreType.DMA,
            pltpu.SemaphoreType.DMA,
        ],
    )(functools.partial(_agg_body, S, D, NCH))
    return grid_kernel(fb, tn, scale)


# speedup vs baseline: 2.7070x; 1.2992x over previous
import functools

import jax
import jax.numpy as jnp
from jax import lax
from jax.experimental import pallas as pl
from jax.experimental.pallas import tpu as pltpu
from jax.experimental.pallas import tpu_sc as plsc

NC = 2
NS = 16
NW = NC * NS
LANES = 16

C = 40
IDX_W = 80
PC = 125
MAGIC = 0x5CBA17E1


def _agg_body(S, D, NCH, N, features_hbm, tn_hbm, scale_hbm,
              out_hbm, ptab_hbm, flags_hbm,
              idx0, idx1, rows0, rows1, out0, out1, scale_v,
              pin0, pin1, pout0, pout1, flv,
              gsem0, gsem1, isem0, isem1, osem0, osem1,
              pisem0, pisem1, posem0, posem1):
    k_slices = (C * S) // IDX_W
    g_regs = D // (2 * LANES)
    npc = N // (NW * PC)

    my_core = lax.axis_index("c")
    sid = lax.axis_index("s")
    wid = sid * NC + my_core

    hi_mask = jnp.full((LANES,), -65536, jnp.int32)
    c7fff = jnp.full((LANES,), 0x7FFF, jnp.int32)
    one = jnp.full((LANES,), 1, jnp.int32)
    sixteen = jnp.full((LANES,), 16, jnp.int32)

    @pl.when(sid == 0)
    def _():
        flv[...] = jnp.zeros((LANES,), jnp.int32)
        pltpu.sync_copy(flv, flags_hbm.at[my_core])

    pltpu.sync_copy(scale_hbm, scale_v)
    scale = scale_v[...]

    pin = (pin0, pin1)
    pout = (pout0, pout1)
    pisem = (pisem0, pisem1)
    posem = (posem0, posem1)
    row_base = wid * (N // NW)

    def pack_in(i, p):
        return pltpu.make_async_copy(
            features_hbm.at[pl.ds(row_base + i * PC, PC)], pin[p], pisem[p])

    def pack_out(i, p):
        return pltpu.make_async_copy(
            pout[p], ptab_hbm.at[pl.ds(row_base + i * PC, PC)], posem[p])

    def pack_compute(p):
        @plsc.parallel_loop(0, PC, unroll=2)
        def rowfn(rr):
            for g in range(g_regs):
                vlo = plsc.bitcast(pin[p][rr, pl.ds(g * LANES, LANES)],
                                   jnp.int32)
                vhi = plsc.bitcast(
                    pin[p][rr, pl.ds(D // 2 + g * LANES, LANES)], jnp.int32)
                rlo = vlo + c7fff + (lax.shift_right_logical(vlo, sixteen)
                                     & one)
                rhi = vhi + c7fff + (lax.shift_right_logical(vhi, sixteen)
                                     & one)
                pout[p][rr, pl.ds(g * LANES, LANES)] = (
                    lax.shift_right_logical(rlo, sixteen)
                    | (rhi & hi_mask))

    pack_in(0, 0).start()
    for i in range(npc):
        p = i % 2
        if i + 1 < npc:
            pack_in(i + 1, 1 - p).start()
        pack_in(i, p).wait()
        if i >= 2:
            pack_out(i - 2, p).wait()
        pack_compute(p)
        pack_out(i, p).start()
    for i in (npc - 2, npc - 1):
        pack_out(i, i % 2).wait()

    plsc.subcore_barrier()

    @pl.when(sid == 0)
    def _():
        flv[...] = jnp.full((LANES,), MAGIC, jnp.int32)
        pltpu.sync_copy(flv, flags_hbm.at[my_core])

        def poll(_):
            pltpu.sync_copy(flags_hbm.at[1 - my_core], flv)
            return jnp.min(flv[...])

        lax.while_loop(lambda v: v != MAGIC, poll, jnp.int32(0))

    plsc.subcore_barrier()

    idx = (idx0, idx1)
    rows = (rows0, rows1)
    outs = (out0, out1)
    gsem = (gsem0, gsem1)
    isem = (isem0, isem1)
    osem = (osem0, osem1)

    def idx_copy(c, p):
        return pltpu.make_async_copy(tn_hbm.at[c], idx[p], isem[p])

    def gather_copies(p):
        return [
            pltpu.make_async_copy(
                ptab_hbm.at[idx[p].at[j]],
                rows[p].at[pl.ds(j * IDX_W, IDX_W)], gsem[p])
            for j in range(k_slices)
        ]

    def out_copy(c, p):
        return pltpu.make_async_copy(outs[p], out_hbm.at[pl.ds(c * C, C)],
                                     osem[p])

    def compute(p):
        @plsc.parallel_loop(0, C, unroll=4)
        def node_body(n):
            r = n * S
            half = D // 2
            for g in range(g_regs):
                sl = pl.ds(g * LANES, LANES)
                acc_lo = None
                acc_hi = None
                for s in range(S):
                    v = rows[p][r + s, sl]
                    lo = plsc.bitcast(lax.shift_left(v, 16), jnp.float32)
                    hi = plsc.bitcast(jnp.bitwise_and(v, hi_mask),
                                      jnp.float32)
                    acc_lo = lo if acc_lo is None else acc_lo + lo
                    acc_hi = hi if acc_hi is None else acc_hi + hi
                outs[p][n, pl.ds(g * LANES, LANES)] = acc_lo * scale
                outs[p][n, pl.ds(half + g * LANES, LANES)] = acc_hi * scale

    @pl.when(wid < NCH)
    def _():
        idx_copy(wid, 0).start()
        idx_copy(wid, 0).wait()
        for cp in gather_copies(0):
            cp.start()

    @pl.when(wid + NW < NCH)
    def _():
        idx_copy(wid + NW, 1).start()

    n_pairs = (-(-NCH // NW) + 1) // 2

    def pair_body(i2, _):
        for p in (0, 1):
            t = 2 * i2 + p
            c = wid + t * NW
            q = 1 - p

            @pl.when(c + NW < NCH)
            def _():
                idx_copy(c + NW, q).wait()
                for cp in gather_copies(q):
                    cp.start()

            @pl.when(c < NCH)
            def _():
                for cp in gather_copies(p):
                    cp.wait()

                @pl.when(c + 2 * NW < NCH)
                def _():
                    idx_copy(c + 2 * NW, p).start()

                @pl.when(i2 >= 1)
                def _():
                    out_copy(c - 2 * NW, p).wait()

                compute(p)
                out_copy(c, p).start()
        return 0

    lax.fori_loop(0, n_pairs, pair_body, 0, unroll=False)

    for p in (0, 1):
        @pl.when(wid + p * NW < NCH)
        def _():
            out_copy(0, p).wait()


def kernel(features, nodes, to_neighs, num_sample):
    del nodes
    B, S = to_neighs.shape
    N, D = features.shape
    assert B % C == 0 and (C * S) % IDX_W == 0 and D % (2 * LANES) == 0
    assert N % (NW * PC) == 0
    NCH = B // C

    tn = to_neighs.reshape(NCH, (C * S) // IDX_W, IDX_W)
    scale = jnp.full((LANES,), 1.0, jnp.float32) / num_sample

    mesh = plsc.VectorSubcoreMesh(
        core_axis_name="c", subcore_axis_name="s",
        num_cores=NC, num_subcores=NS)
    k_sl = (C * S) // IDX_W
    grid_kernel = functools.partial(
        pl.kernel,
        out_type=(
            jax.ShapeDtypeStruct((B, D), jnp.float32),
            jax.ShapeDtypeStruct((N, D // 2), jnp.int32),
            jax.ShapeDtypeStruct((NC, LANES), jnp.int32),
        ),
        mesh=mesh,
        compiler_params=pltpu.CompilerParams(
            use_tc_tiling_on_sc=False, needs_layout_passes=False),
        scratch_types=[
            pltpu.VMEM((k_sl, IDX_W), jnp.int32),
            pltpu.VMEM((k_sl, IDX_W), jnp.int32),
            pltpu.VMEM((C * S, D // 2), jnp.int32),
            pltpu.VMEM((C * S, D // 2), jnp.int32),
            pltpu.VMEM((C, D), jnp.float32),
            pltpu.VMEM((C, D), jnp.float32),
            pltpu.VMEM((LANES,), jnp.float32),
            pltpu.VMEM((PC, D), jnp.float32),
            pltpu.VMEM((PC, D), jnp.float32),
            pltpu.VMEM((PC, D // 2), jnp.int32),
            pltpu.VMEM((PC, D // 2), jnp.int32),
            pltpu.VMEM((LANES,), jnp.int32),
            pltpu.SemaphoreType.DMA,
            pltpu.SemaphoreType.DMA,
            pltpu.SemaphoreType.DMA,
            pltpu.SemaphoreType.DMA,
            pltpu.SemaphoreType.DMA,
            pltpu.SemaphoreType.DMA,
            pltpu.SemaphoreType.DMA,
            pltpu.SemaphoreType.DMA,
            pltpu.SemaphoreType.DMA,
            pltpu.SemaphoreType.DMA,
        ],
    )(functools.partial(_agg_body, S, D, NCH, N))
    out, _ptab, _flags = grid_kernel(features, tn, scale)
    return out
